# baseline (speedup 1.0000x reference)
import jax
import jax.numpy as jnp
from jax.experimental import pallas as pl
from jax.experimental.pallas import tpu as pltpu

N_REVIEW = 100000
D_REVIEW = 799
H = 128
BM = 1000


def _fused_body(x_ref, w1_ref, b1_ref,
                w2st_ref, b2st_ref, w2sf_ref, b2sf_ref, w2sb_ref, b2sb_ref,
                wr_ref, br_ref, wi_ref, bi_ref, ws_ref, bs_ref,
                out_r_ref, out_i_ref, out_s_ref):
    x = x_ref[...]
    a = jnp.dot(x, w1_ref[...], preferred_element_type=jnp.float32)
    a = jnp.maximum(a + b1_ref[...], 0.0)
    x1_review = a[:, :H]
    x1_product = a[:, H:]

    x2r = jnp.maximum(
        jnp.dot(x1_review, w2st_ref[...], preferred_element_type=jnp.float32)
        + b2st_ref[...], 0.0)
    x2i = jnp.maximum(
        jnp.dot(x1_review, w2sf_ref[...], preferred_element_type=jnp.float32)
        + b2sf_ref[...], 0.0)
    x2s = jnp.maximum(
        jnp.dot(x1_product, w2sb_ref[...], preferred_element_type=jnp.float32)
        + b2sb_ref[...], 0.0)

    out_r_ref[...] = jax.nn.sigmoid(
        jnp.dot(x2r, wr_ref[...], preferred_element_type=jnp.float32)
        + br_ref[...])
    out_i_ref[...] = jax.nn.sigmoid(
        jnp.dot(x2i, wi_ref[...], preferred_element_type=jnp.float32)
        + bi_ref[...])
    out_s_ref[...] = jax.nn.sigmoid(
        jnp.dot(x2s, ws_ref[...], preferred_element_type=jnp.float32)
        + bs_ref[...])


def kernel(x_review, x_product, edge_written_for, edge_sold_by, edge_sent_from,
           edge_similar_to,
           W1_wf, b1_wf, W1_sb, b1_sb, W1_sf, b1_sf, W1_st, b1_st,
           W2_wf, b2_wf, W2_sb, b2_sb, W2_sf, b2_sf, W2_st, b2_st,
           Wr, br, Wi, bi, Ws, bs):
    w1 = jnp.concatenate([W1_st, W1_wf], axis=1)
    b1 = jnp.concatenate([b1_st, b1_wf])[None, :]

    full = lambda shape: pl.BlockSpec(shape, lambda i: (0, 0))
    grid = N_REVIEW // BM

    out_r, out_i, out_s = pl.pallas_call(
        _fused_body,
        grid=(grid,),
        in_specs=[
            pl.BlockSpec((BM, D_REVIEW), lambda i: (i, 0)),
            full((D_REVIEW, 2 * H)), full((1, 2 * H)),
            full((H, H)), full((1, H)),
            full((H, H)), full((1, H)),
            full((H, H)), full((1, H)),
            full((H, 1)), full((1, 1)),
            full((H, 1)), full((1, 1)),
            full((H, 1)), full((1, 1)),
        ],
        out_specs=[
            pl.BlockSpec((BM, 1), lambda i: (i, 0)),
            pl.BlockSpec((BM, 1), lambda i: (i, 0)),
            pl.BlockSpec((BM, 1), lambda i: (i, 0)),
        ],
        out_shape=[
            jax.ShapeDtypeStruct((N_REVIEW, 1), jnp.float32),
            jax.ShapeDtypeStruct((N_REVIEW, 1), jnp.float32),
            jax.ShapeDtypeStruct((N_REVIEW, 1), jnp.float32),
        ],
        compiler_params=pltpu.CompilerParams(
            dimension_semantics=("arbitrary",),
        ),
    )(x_review, w1, b1,
      W2_st, b2_st[None, :], W2_sf, b2_sf[None, :], W2_sb, b2_sb[None, :],
      Wr, br[None, :], Wi, bi[None, :], Ws, bs[None, :])

    return (out_r[:, 0], out_i[:, 0], out_s[:, 0])
